# Initial kernel scaffold; baseline (speedup 1.0000x reference)
import jax
import jax.numpy as jnp
from jax.experimental import pallas as pl


def kernel(x, threshold):
    raise NotImplementedError("write your pallas kernel here")


# speedup vs baseline: 1.3011x; 1.3011x over previous
import jax
import jax.numpy as jnp
from jax import lax
from jax.experimental import pallas as pl
from jax.experimental.pallas import tpu as pltpu

_K = 64


def _tc_body(thr_ref, x_ref, feat_ref, idx_ref, valid_ref):
    x = x_ref[0]
    thr = thr_ref[0]
    Q, N = x.shape
    iota = lax.broadcasted_iota(jnp.int32, (Q, N), 1)
    valid = x >= thr
    valid_ref[0] = valid.astype(jnp.int8)
    vals0 = jnp.where(valid, x, 0.0)
    feat0 = jnp.zeros((Q, N), jnp.float32)
    inds0 = jnp.zeros((Q, _K), jnp.int32)

    def body(r, carry):
        vals, feat, inds = carry
        m = jnp.max(vals, axis=1, keepdims=True)
        eq = vals == m
        pos = jnp.min(jnp.where(eq, iota, jnp.int32(N)), axis=1, keepdims=True)
        onehot = iota == pos
        feat = jnp.where(onehot, 1.0, feat)
        vals = jnp.where(onehot, -1.0, vals)
        inds = jnp.where(
            lax.broadcasted_iota(jnp.int32, (1, _K), 1) == r, pos, inds)
        return vals, feat, inds

    vals, feat, inds = lax.fori_loop(0, _K, body, (vals0, feat0, inds0))
    feat_ref[0] = feat
    idx_ref[0] = inds


def kernel(x, threshold):
    B, Q, N = x.shape
    thr = jnp.reshape(threshold.astype(jnp.float32), (1,))
    feat, inds, valid = pl.pallas_call(
        _tc_body,
        grid=(B,),
        in_specs=[
            pl.BlockSpec(memory_space=pltpu.SMEM),
            pl.BlockSpec((1, Q, N), lambda b: (b, 0, 0)),
        ],
        out_specs=[
            pl.BlockSpec((1, Q, N), lambda b: (b, 0, 0)),
            pl.BlockSpec((1, Q, _K), lambda b: (b, 0, 0)),
            pl.BlockSpec((1, Q, N), lambda b: (b, 0, 0)),
        ],
        out_shape=[
            jax.ShapeDtypeStruct((B, Q, N), jnp.float32),
            jax.ShapeDtypeStruct((B, Q, _K), jnp.int32),
            jax.ShapeDtypeStruct((B, Q, N), jnp.int8),
        ],
    )(thr, x)
    return feat, inds, valid.astype(bool)


# speedup vs baseline: 13.1397x; 10.0986x over previous
import functools

import jax
import jax.numpy as jnp
from jax import lax
from jax.experimental import pallas as pl
from jax.experimental.pallas import tpu as pltpu
from jax.experimental.pallas import tpu_sc as plsc

_K = 64
_N = 8192
_Q = 8
_B = 32
_ROWS = _B * _Q
_NW = 32
_RPW = _ROWS // _NW
_G = 64
_GSZ = _N // _G
_GV = _GSZ // 16


_GDN = lax.GatherDimensionNumbers(
    offset_dims=(), collapsed_slice_dims=(0,), start_index_map=(0,))


def _shuf(v, idx):
    return lax.gather(v, idx[:, None], _GDN, (1,),
                      mode=lax.GatherScatterMode.PROMISE_IN_BOUNDS)


def _sc_body(x_hbm, thr_hbm, feat_hbm, idx_hbm, vals_v, feat_v, gmax_v,
             oidx_v, thr_v):
    wid = lax.axis_index("s") * 2 + lax.axis_index("c")
    lane = lax.iota(jnp.int32, 16)
    lane0 = lane == 0
    ones16 = jnp.full((16,), 1.0, jnp.float32)
    zeros16 = jnp.zeros((16,), jnp.float32)
    neg16 = jnp.full((16,), -1.0, jnp.float32)
    xors = [lane ^ 1, lane ^ 2, lane ^ 4, lane ^ 8]

    def vmax_all(v):
        for xi in xors:
            v = jnp.maximum(v, _shuf(v, xi))
        return v

    def vmin_all(v):
        for xi in xors:
            v = jnp.minimum(v, _shuf(v, xi))
        return v

    pltpu.sync_copy(thr_hbm, thr_v)
    thrv = thr_v[pl.ds(0, 16)]

    def zf(i, c):
        feat_v[pl.ds(i * 16, 16)] = zeros16
        return c

    lax.fori_loop(0, _N // 16, zf, 0)

    def row_body(j, carry):
        row = wid * _RPW + j
        pltpu.sync_copy(x_hbm.at[row], vals_v)

        def build(g, c):
            base = g * _GSZ
            gm = None
            for t in range(_GV):
                v = vals_v[pl.ds(base + t * 16, 16)]
                v = jnp.where(v >= thrv, v, zeros16)
                vals_v[pl.ds(base + t * 16, 16)] = v
                gm = v if gm is None else jnp.maximum(gm, v)
            gmv = vmax_all(gm)
            plsc.store_scatter(gmax_v, [jnp.full((16,), g, jnp.int32)],
                               gmv, mask=lane0)
            return c

        lax.fori_loop(0, _G, build, 0)

        def ext(r, c):
            g0 = gmax_v[pl.ds(0, 16)]
            g1 = gmax_v[pl.ds(16, 16)]
            g2 = gmax_v[pl.ds(32, 16)]
            g3 = gmax_v[pl.ds(48, 16)]
            m = jnp.maximum(jnp.maximum(g0, g1), jnp.maximum(g2, g3))
            big = vmax_all(m)
            c0 = jnp.where(g0 == big, lane, 64)
            c1 = jnp.where(g1 == big, lane + 16, 64)
            c2 = jnp.where(g2 == big, lane + 32, 64)
            c3 = jnp.where(g3 == big, lane + 48, 64)
            gsv = vmin_all(jnp.minimum(jnp.minimum(c0, c1),
                                       jnp.minimum(c2, c3)))
            basev = gsv * _GSZ
            pc = None
            idxvs = []
            for t in range(_GV):
                iv = basev + lane + t * 16
                idxvs.append(iv)
                v = plsc.load_gather(vals_v, [iv])
                cd = jnp.where(v == big, iv, _N)
                pc = cd if pc is None else jnp.minimum(pc, cd)
            pvec = vmin_all(pc)
            plsc.store_scatter(oidx_v, [jnp.full((16,), j * _K + r,
                                          jnp.int32)], pvec, mask=lane0)
            plsc.store_scatter(feat_v, [pvec], ones16, mask=lane0)
            plsc.store_scatter(vals_v, [pvec], neg16, mask=lane0)
            gm = None
            for iv in idxvs:
                v = plsc.load_gather(vals_v, [iv])
                gm = v if gm is None else jnp.maximum(gm, v)
            gmv = vmax_all(gm)
            plsc.store_scatter(gmax_v, [gsv], gmv, mask=lane0)
            return c

        lax.fori_loop(0, _K, ext, 0)

        pltpu.sync_copy(feat_v, feat_hbm.at[row])
        for t in range(_K // 16):
            idxv = oidx_v[pl.ds(j * _K + t * 16, 16)]
            plsc.store_scatter(feat_v, [idxv], zeros16)
        return carry

    lax.fori_loop(0, _RPW, row_body, 0)
    pltpu.sync_copy(oidx_v, idx_hbm.at[pl.ds(wid * _RPW * _K, _RPW * _K)])


@jax.jit
def _sc_call(x2, thr16):
    mesh = plsc.VectorSubcoreMesh(core_axis_name="c", subcore_axis_name="s")
    f = functools.partial(
        pl.kernel,
        out_type=[
            jax.ShapeDtypeStruct((_ROWS, _N), jnp.float32),
            jax.ShapeDtypeStruct((_ROWS * _K,), jnp.int32),
        ],
        mesh=mesh,
        scratch_types=[
            pltpu.VMEM((_N,), jnp.float32),
            pltpu.VMEM((_N,), jnp.float32),
            pltpu.VMEM((128,), jnp.float32),
            pltpu.VMEM((_RPW * _K,), jnp.int32),
            pltpu.VMEM((128,), jnp.float32),
        ],
        compiler_params=pltpu.CompilerParams(needs_layout_passes=False),
    )(_sc_body)
    return f(x2, thr16)


def _tc_valid_body(thr_ref, x_ref, valid_ref):
    valid_ref[...] = (x_ref[...] >= thr_ref[0]).astype(jnp.int8)


def kernel(x, threshold):
    B, Q, N = x.shape
    thr16 = jnp.broadcast_to(threshold.astype(jnp.float32), (128,))
    x2 = x.reshape(B * Q, N)
    feat2, inds1 = _sc_call(x2, thr16)
    inds2 = inds1.reshape(B * Q, _K)
    valid = pl.pallas_call(
        _tc_valid_body,
        grid=(B,),
        in_specs=[
            pl.BlockSpec(memory_space=pltpu.SMEM),
            pl.BlockSpec((1, Q, N), lambda b: (b, 0, 0)),
        ],
        out_specs=pl.BlockSpec((1, Q, N), lambda b: (b, 0, 0)),
        out_shape=jax.ShapeDtypeStruct((B, Q, N), jnp.int8),
    )(jnp.reshape(threshold.astype(jnp.float32), (1,)), x)
    return (feat2.reshape(B, Q, N), inds2.reshape(B, Q, _K),
            valid.astype(bool))


# speedup vs baseline: 17.9132x; 1.3633x over previous
import functools

import jax
import jax.numpy as jnp
from jax import lax
from jax.experimental import pallas as pl
from jax.experimental.pallas import tpu as pltpu
from jax.experimental.pallas import tpu_sc as plsc

_K = 64
_N = 8192
_Q = 8
_B = 32
_ROWS = _B * _Q
_NW = 32
_RPW = _ROWS // _NW
_G = 64
_GSZ = _N // _G
_GV = _GSZ // 16


_GDN = lax.GatherDimensionNumbers(
    offset_dims=(), collapsed_slice_dims=(0,), start_index_map=(0,))


def _shuf(v, idx):
    return lax.gather(v, idx[:, None], _GDN, (1,),
                      mode=lax.GatherScatterMode.PROMISE_IN_BOUNDS)


def _sc_body(x_hbm, thr_hbm, feat_hbm, idx_hbm, vals_a, vals_b, feat_a,
             feat_b, oidx_v, thr_v):
    wid = lax.axis_index("s") * 2 + lax.axis_index("c")
    lane = lax.iota(jnp.int32, 16)
    lane0 = lane == 0
    ones16 = jnp.full((16,), 1.0, jnp.float32)
    zeros16 = jnp.zeros((16,), jnp.float32)
    neg16 = jnp.full((16,), -1.0, jnp.float32)
    xors = [lane ^ 1, lane ^ 2, lane ^ 4, lane ^ 8]

    def vmax_all(v):
        for xi in xors:
            v = jnp.maximum(v, _shuf(v, xi))
        return v

    def vmin_all(v):
        for xi in xors:
            v = jnp.minimum(v, _shuf(v, xi))
        return v

    pltpu.sync_copy(thr_hbm, thr_v)
    thrv = thr_v[pl.ds(0, 16)]

    def zf(i, c):
        feat_a[pl.ds(i * 16, 16)] = zeros16
        feat_b[pl.ds(i * 16, 16)] = zeros16
        return c

    lax.fori_loop(0, _N // 16, zf, 0)

    def pair_body(jj, carry):
        ja = jj * 2
        jb = ja + 1
        rowa = wid * _RPW + ja
        rowb = rowa + 1
        pltpu.sync_copy(x_hbm.at[rowa], vals_a)
        pltpu.sync_copy(x_hbm.at[rowb], vals_b)

        def build(g, gc):
            base = g * _GSZ
            gmod = lane == (g & 15)
            gdiv = g // 16
            out = list(gc)
            for which, vref in ((0, vals_a), (1, vals_b)):
                gm = None
                for t in range(_GV):
                    v = vref[pl.ds(base + t * 16, 16)]
                    v = jnp.where(v >= thrv, v, zeros16)
                    vref[pl.ds(base + t * 16, 16)] = v
                    gm = v if gm is None else jnp.maximum(gm, v)
                gmv = vmax_all(gm)
                for xq in range(4):
                    k = which * 4 + xq
                    out[k] = jnp.where(gmod & (gdiv == xq), gmv, out[k])
            return tuple(out)

        gcar = lax.fori_loop(0, _G, build, (zeros16,) * 8)

        def ext(r, gc):
            out = list(gc)
            for which, vref, fref, jrow in ((0, vals_a, feat_a, ja),
                                            (1, vals_b, feat_b, jb)):
                g0, g1, g2, g3 = out[which * 4:which * 4 + 4]
                mm = jnp.maximum(jnp.maximum(g0, g1), jnp.maximum(g2, g3))
                big = vmax_all(mm)
                c0 = jnp.where(g0 == big, lane, 64)
                c1 = jnp.where(g1 == big, lane + 16, 64)
                c2 = jnp.where(g2 == big, lane + 32, 64)
                c3 = jnp.where(g3 == big, lane + 48, 64)
                gsv = vmin_all(jnp.minimum(jnp.minimum(c0, c1),
                                           jnp.minimum(c2, c3)))
                basev = gsv * _GSZ
                pc = None
                ivs = []
                vts = []
                for t in range(_GV):
                    iv = basev + lane + t * 16
                    v = plsc.load_gather(vref, [iv])
                    ivs.append(iv)
                    vts.append(v)
                    cd = jnp.where(v == big, iv, _N)
                    pc = cd if pc is None else jnp.minimum(pc, cd)
                pvec = vmin_all(pc)
                plsc.store_scatter(oidx_v, [jnp.full((16,), jrow * _K,
                                            jnp.int32) + r], pvec,
                                   mask=lane0)
                plsc.store_scatter(fref, [pvec], ones16, mask=lane0)
                plsc.store_scatter(vref, [pvec], neg16, mask=lane0)
                gm = None
                for iv, v in zip(ivs, vts):
                    vk = jnp.where(iv == pvec, neg16, v)
                    gm = vk if gm is None else jnp.maximum(gm, vk)
                gmv = vmax_all(gm)
                gmod = lane == (gsv & 15)
                gdiv = gsv >> 4
                for xq in range(4):
                    k = which * 4 + xq
                    out[k] = jnp.where(gmod & (gdiv == xq), gmv, out[k])
            return tuple(out)

        lax.fori_loop(0, _K, ext, gcar)

        pltpu.sync_copy(feat_a, feat_hbm.at[rowa])
        pltpu.sync_copy(feat_b, feat_hbm.at[rowb])
        for jrow, fref in ((ja, feat_a), (jb, feat_b)):
            for t in range(_K // 16):
                idxv = oidx_v[pl.ds(jrow * _K + t * 16, 16)]
                plsc.store_scatter(fref, [idxv], zeros16)
        return carry

    lax.fori_loop(0, _RPW // 2, pair_body, 0)
    pltpu.sync_copy(oidx_v, idx_hbm.at[pl.ds(wid * _RPW * _K, _RPW * _K)])


@jax.jit
def _sc_call(x2, thr16):
    mesh = plsc.VectorSubcoreMesh(core_axis_name="c", subcore_axis_name="s")
    f = functools.partial(
        pl.kernel,
        out_type=[
            jax.ShapeDtypeStruct((_ROWS, _N), jnp.float32),
            jax.ShapeDtypeStruct((_ROWS * _K,), jnp.int32),
        ],
        mesh=mesh,
        scratch_types=[
            pltpu.VMEM((_N,), jnp.float32),
            pltpu.VMEM((_N,), jnp.float32),
            pltpu.VMEM((_N,), jnp.float32),
            pltpu.VMEM((_N,), jnp.float32),
            pltpu.VMEM((_RPW * _K,), jnp.int32),
            pltpu.VMEM((128,), jnp.float32),
        ],
        compiler_params=pltpu.CompilerParams(needs_layout_passes=False),
    )(_sc_body)
    return f(x2, thr16)


def _tc_valid_body(thr_ref, x_ref, valid_ref):
    valid_ref[...] = (x_ref[...] >= thr_ref[0]).astype(jnp.int8)


def kernel(x, threshold):
    B, Q, N = x.shape
    thr16 = jnp.broadcast_to(threshold.astype(jnp.float32), (128,))
    x2 = x.reshape(B * Q, N)
    feat2, inds1 = _sc_call(x2, thr16)
    inds2 = inds1.reshape(B * Q, _K)
    valid = pl.pallas_call(
        _tc_valid_body,
        grid=(B,),
        in_specs=[
            pl.BlockSpec(memory_space=pltpu.SMEM),
            pl.BlockSpec((1, Q, N), lambda b: (b, 0, 0)),
        ],
        out_specs=pl.BlockSpec((1, Q, N), lambda b: (b, 0, 0)),
        out_shape=jax.ShapeDtypeStruct((B, Q, N), jnp.int8),
    )(jnp.reshape(threshold.astype(jnp.float32), (1,)), x)
    return (feat2.reshape(B, Q, N), inds2.reshape(B, Q, _K),
            valid.astype(bool))
